# baseline (speedup 1.0000x reference)
import functools

import jax
import jax.numpy as jnp
from jax import lax
from jax.experimental import pallas as pl
from jax.experimental.pallas import tpu as pltpu
from jax.experimental.pallas import tpu_sc as plsc

_N = 10000
_D = 128
_K = 16
_E = 320000
_EG = 160000

_SELU_ALPHA = 1.6732632423543772
_SELU_SCALE = 1.0507009873554805


_BLK = 1000
_NSTEP = _N // _BLK


def _tc_body(x_ref, w_ref, b_ref, e_ref, g1_ref, g2_ref,
             a_ref, csum_ref, pooled_ref,
             er_ref, ec_ref, g1r_ref, g1c_ref, g2r_ref, g2c_ref,
             atx_ref, esem):
    i = pl.program_id(0)

    pairs = ((e_ref, er_ref, ec_ref), (g1_ref, g1r_ref, g1c_ref),
             (g2_ref, g2r_ref, g2c_ref))

    @pl.when(i == 0)
    def _():
        for src, dr, dc in pairs:
            pltpu.make_async_copy(src.at[0], dr, esem).start()
            pltpu.make_async_copy(src.at[1], dc, esem).start()

    @pl.when(i == _NSTEP - 1)
    def _():
        for src, dr, dc in pairs:
            pltpu.make_async_copy(src.at[0], dr, esem).wait()
            pltpu.make_async_copy(src.at[1], dc, esem).wait()
    x = x_ref[...]
    logits = jnp.dot(x, w_ref[...], preferred_element_type=jnp.float32)
    logits = logits + b_ref[...]
    mx = jnp.max(logits, axis=1, keepdims=True)
    e = jnp.exp(logits - mx)
    a = e / jnp.sum(e, axis=1, keepdims=True)
    a_ref[...] = a

    @pl.when(i == 0)
    def _():
        csum_ref[...] = jnp.zeros_like(csum_ref)
        atx_ref[...] = jnp.zeros_like(atx_ref)

    csum_ref[...] += jnp.sum(a, axis=0, keepdims=True)
    atx_ref[...] += lax.dot_general(
        x, a, (((0,), (0,)), ((), ())), preferred_element_type=jnp.float32)

    @pl.when(i == _NSTEP - 1)
    def _():
        pooled = atx_ref[...] / csum_ref[...]
        pooled_ref[...] = _SELU_SCALE * jnp.where(
            pooled > 0, pooled, _SELU_ALPHA * (jnp.exp(pooled) - 1.0))


_tc_call = pl.pallas_call(
    _tc_body,
    grid=(_NSTEP,),
    in_specs=[
        pl.BlockSpec((_BLK, _D), lambda i: (i, 0)),
        pl.BlockSpec((_D, _K), lambda i: (0, 0)),
        pl.BlockSpec((1, _K), lambda i: (0, 0)),
        pl.BlockSpec(memory_space=pl.ANY),
        pl.BlockSpec(memory_space=pl.ANY),
        pl.BlockSpec(memory_space=pl.ANY),
    ],
    out_specs=[
        pl.BlockSpec((_BLK, _K), lambda i: (i, 0)),
        pl.BlockSpec((1, _K), lambda i: (0, 0)),
        pl.BlockSpec((_D, _K), lambda i: (0, 0)),
        pl.BlockSpec(memory_space=pl.ANY),
        pl.BlockSpec(memory_space=pl.ANY),
        pl.BlockSpec(memory_space=pl.ANY),
        pl.BlockSpec(memory_space=pl.ANY),
        pl.BlockSpec(memory_space=pl.ANY),
        pl.BlockSpec(memory_space=pl.ANY),
    ],
    out_shape=[
        jax.ShapeDtypeStruct((_N, _K), jnp.float32),
        jax.ShapeDtypeStruct((1, _K), jnp.float32),
        jax.ShapeDtypeStruct((_D, _K), jnp.float32),
        jax.ShapeDtypeStruct((_E,), jnp.int32),
        jax.ShapeDtypeStruct((_E,), jnp.int32),
        jax.ShapeDtypeStruct((_EG,), jnp.int32),
        jax.ShapeDtypeStruct((_EG,), jnp.int32),
        jax.ShapeDtypeStruct((_EG,), jnp.int32),
        jax.ShapeDtypeStruct((_EG,), jnp.int32),
    ],
    scratch_shapes=[pltpu.VMEM((_D, _K), jnp.float32),
                    pltpu.SemaphoreType.DMA],
)


_NC = 2
_NS = 16
_NW = _NC * _NS
_C = 1000
_CH_MAIN = _E // _NW // _C
_CH_G = _EG // _NW // _C
_NCH = _CH_MAIN + 2 * _CH_G
_U = 20
_NACC = 2


def _edge_body(a_hbm, er_hbm, ec_hbm, g1r_hbm, g1c_hbm, g2r_hbm, g2c_hbm,
               out_hbm,
               ridx0, cidx0, ridx1, cidx1,
               rbuf0, cbuf0, rbuf1, cbuf1,
               outv, semi0, semi1, semg0, semg1):
    wid = lax.axis_index("s") * _NC + lax.axis_index("c")

    idx = ((ridx0, cidx0), (ridx1, cidx1))
    buf = ((rbuf0, cbuf0), (rbuf1, cbuf1))
    semi = (semi0, semi1)
    semg = (semg0, semg1)

    chunks = (
        [(er_hbm, ec_hbm, wid * (_E // _NW) + i * _C, 0)
         for i in range(_CH_MAIN)]
        + [(g1r_hbm, g1c_hbm, wid * (_EG // _NW) + i * _C, 1)
           for i in range(_CH_G)]
        + [(g2r_hbm, g2c_hbm, wid * (_EG // _NW) + i * _C, 2)
           for i in range(_CH_G)]
    )

    def stage_idx(k, slot):
        rref, cref, off, _ = chunks[k]
        c1 = pltpu.async_copy(rref.at[pl.ds(off, _C)], idx[slot][0],
                              semi[slot])
        c2 = pltpu.async_copy(cref.at[pl.ds(off, _C)], idx[slot][1],
                              semi[slot])
        return c1, c2

    def fire_gather(slot):
        c1 = pltpu.async_copy(a_hbm.at[idx[slot][0]], buf[slot][0],
                              semg[slot])
        c2 = pltpu.async_copy(a_hbm.at[idx[slot][1]], buf[slot][1],
                              semg[slot])
        return c1, c2

    pis = {0: stage_idx(0, 0), 1: stage_idx(1, 1)}
    pis[0][0].wait()
    pis[0][1].wait()
    pgs = {0: fire_gather(0)}

    zero = jnp.zeros((16,), jnp.float32)
    accs = [tuple(zero for _ in range(2 * _NACC)) for _ in range(3)]
    for k in range(_NCH):
        slot = k % 2
        if k + 1 < _NCH:
            pis[k + 1][0].wait()
            pis[k + 1][1].wait()
            pgs[k + 1] = fire_gather((k + 1) % 2)
        pgs[k][0].wait()
        pgs[k][1].wait()
        if k + 2 < _NCH:
            pis[k + 2] = stage_idx(k + 2, slot)
        rb, cb = buf[slot]

        def body(jj, carry, rb=rb, cb=cb):
            acc = list(carry)
            j0 = jj * _U
            for u in range(_U):
                r = rb[j0 + u]
                c = cb[j0 + u]
                i = 2 * (u % _NACC)
                acc[i] = acc[i] + r * c
                acc[i + 1] = acc[i + 1] + c
            return tuple(acc)

        s = chunks[k][3]
        accs[s] = lax.fori_loop(0, _C // _U, body, accs[s])

    for s in range(3):
        ad = accs[s][0] + accs[s][2]
        ac = accs[s][1] + accs[s][3]
        outv[pl.ds(32 * s, 16)] = ad
        outv[pl.ds(32 * s + 16, 16)] = ac
    pltpu.sync_copy(outv, out_hbm.at[pl.ds(wid * 96, 96)])


_edge_kernel = pl.kernel(
    _edge_body,
    out_type=jax.ShapeDtypeStruct((_NW * 96,), jnp.float32),
    mesh=plsc.VectorSubcoreMesh(core_axis_name="c", subcore_axis_name="s"),
    compiler_params=pltpu.CompilerParams(use_tc_tiling_on_sc=False),
    scratch_types=[
        pltpu.VMEM((_C,), jnp.int32),
        pltpu.VMEM((_C,), jnp.int32),
        pltpu.VMEM((_C,), jnp.int32),
        pltpu.VMEM((_C,), jnp.int32),
        pltpu.VMEM((_C, _K), jnp.float32),
        pltpu.VMEM((_C, _K), jnp.float32),
        pltpu.VMEM((_C, _K), jnp.float32),
        pltpu.VMEM((_C, _K), jnp.float32),
        pltpu.VMEM((96,), jnp.float32),
        pltpu.SemaphoreType.DMA,
        pltpu.SemaphoreType.DMA,
        pltpu.SemaphoreType.DMA,
        pltpu.SemaphoreType.DMA,
    ],
)


def kernel(features, edge_index, g1_edge_index, g2_edge_index, lamda, W, b):
    n = features.shape[0]
    k = W.shape[1]
    m = float(edge_index.shape[1])
    m1 = float(g1_edge_index.shape[1])
    m2 = float(g2_edge_index.shape[1])

    (a, csum, pooled_t, er, ec, g1r, g1c, g2r, g2c) = _tc_call(
        features, W, b.reshape(1, _K),
        edge_index, g1_edge_index, g2_edge_index)
    assignments = a
    cs = csum.reshape(_K)
    features_pooled = pooled_t.T

    partials = _edge_kernel(
        a, er, ec, g1r, g1c, g2r, g2c).reshape(_NW, 3, 2, _K)

    def set_stats(s):
        dot = jnp.sum(partials[:, s, 0, :])
        v = jnp.sum(partials[:, s, 1, :], axis=0)
        return dot, jnp.sum(v ** 2)

    s_main, vsq_main = set_stats(0)
    s_g1, vsq_g1 = set_stats(1)
    s_g2, vsq_g2 = set_stats(2)

    spectral_loss = -(s_main - vsq_main / (2.0 * m)) / (2.0 * m)
    q_g1 = (s_g1 - vsq_g1 / (2.0 * m1)) / (2.0 * m)
    q_g2 = (s_g2 - vsq_g2 / (2.0 * m2)) / (2.0 * m)
    edge_ratio = m / ((m1 + m2) / 2.0 + 1e-8)
    fairness_loss = edge_ratio * (-jnp.minimum(q_g1, q_g2))
    collapse_loss = (jnp.linalg.norm(cs) / n * jnp.sqrt(float(k)) - 1.0)

    total_loss = (spectral_loss + lamda * fairness_loss
                  + 0.1 * collapse_loss)
    return features_pooled, assignments, total_loss


# speedup vs baseline: 2.2674x; 2.2674x over previous
import functools

import jax
import jax.numpy as jnp
from jax import lax
from jax.experimental import pallas as pl
from jax.experimental.pallas import tpu as pltpu
from jax.experimental.pallas import tpu_sc as plsc

_N = 10000
_D = 128
_K = 16
_E = 320000
_EG = 160000

_SELU_ALPHA = 1.6732632423543772
_SELU_SCALE = 1.0507009873554805


_BLK = 1000
_NSTEP = _N // _BLK


def _tc_body(x_ref, w_ref, b_ref, e_ref, g1_ref, g2_ref,
             a_ref, csum_ref, pooled_ref,
             er_ref, ec_ref, g1r_ref, g1c_ref, g2r_ref, g2c_ref,
             atx_ref):
    i = pl.program_id(0)

    @pl.when(i == 0)
    def _():
        er_ref[...] = e_ref[0, :]
        ec_ref[...] = e_ref[1, :]
        g1r_ref[...] = g1_ref[0, :]
        g1c_ref[...] = g1_ref[1, :]
        g2r_ref[...] = g2_ref[0, :]
        g2c_ref[...] = g2_ref[1, :]
    x = x_ref[...]
    logits = jnp.dot(x, w_ref[...], preferred_element_type=jnp.float32)
    logits = logits + b_ref[...]
    mx = jnp.max(logits, axis=1, keepdims=True)
    e = jnp.exp(logits - mx)
    a = e / jnp.sum(e, axis=1, keepdims=True)
    a_ref[...] = a

    @pl.when(i == 0)
    def _():
        csum_ref[...] = jnp.zeros_like(csum_ref)
        atx_ref[...] = jnp.zeros_like(atx_ref)

    csum_ref[...] += jnp.sum(a, axis=0, keepdims=True)
    atx_ref[...] += lax.dot_general(
        x, a, (((0,), (0,)), ((), ())), preferred_element_type=jnp.float32)

    @pl.when(i == _NSTEP - 1)
    def _():
        pooled = atx_ref[...] / csum_ref[...]
        pooled_ref[...] = _SELU_SCALE * jnp.where(
            pooled > 0, pooled, _SELU_ALPHA * (jnp.exp(pooled) - 1.0))


_tc_call = pl.pallas_call(
    _tc_body,
    grid=(_NSTEP,),
    in_specs=[
        pl.BlockSpec((_BLK, _D), lambda i: (i, 0)),
        pl.BlockSpec((_D, _K), lambda i: (0, 0)),
        pl.BlockSpec((1, _K), lambda i: (0, 0)),
        pl.BlockSpec((2, _E), lambda i: (0, 0)),
        pl.BlockSpec((2, _EG), lambda i: (0, 0)),
        pl.BlockSpec((2, _EG), lambda i: (0, 0)),
    ],
    out_specs=[
        pl.BlockSpec((_BLK, _K), lambda i: (i, 0)),
        pl.BlockSpec((1, _K), lambda i: (0, 0)),
        pl.BlockSpec((_D, _K), lambda i: (0, 0)),
        pl.BlockSpec((_E,), lambda i: (0,)),
        pl.BlockSpec((_E,), lambda i: (0,)),
        pl.BlockSpec((_EG,), lambda i: (0,)),
        pl.BlockSpec((_EG,), lambda i: (0,)),
        pl.BlockSpec((_EG,), lambda i: (0,)),
        pl.BlockSpec((_EG,), lambda i: (0,)),
    ],
    out_shape=[
        jax.ShapeDtypeStruct((_N, _K), jnp.float32),
        jax.ShapeDtypeStruct((1, _K), jnp.float32),
        jax.ShapeDtypeStruct((_D, _K), jnp.float32),
        jax.ShapeDtypeStruct((_E,), jnp.int32),
        jax.ShapeDtypeStruct((_E,), jnp.int32),
        jax.ShapeDtypeStruct((_EG,), jnp.int32),
        jax.ShapeDtypeStruct((_EG,), jnp.int32),
        jax.ShapeDtypeStruct((_EG,), jnp.int32),
        jax.ShapeDtypeStruct((_EG,), jnp.int32),
    ],
    scratch_shapes=[pltpu.VMEM((_D, _K), jnp.float32)],
)


_NC = 2
_NS = 16
_NW = _NC * _NS
_C = 1000
_CH_MAIN = _E // _NW // _C
_CH_G = _EG // _NW // _C
_NCH = _CH_MAIN + 2 * _CH_G
_U = 20
_NACC = 2


def _edge_body(a_hbm, er_hbm, ec_hbm, g1r_hbm, g1c_hbm, g2r_hbm, g2c_hbm,
               out_hbm,
               ridx0, cidx0, ridx1, cidx1,
               rbuf0, cbuf0, rbuf1, cbuf1,
               outv, semi0, semi1, semg0, semg1):
    wid = lax.axis_index("s") * _NC + lax.axis_index("c")

    idx = ((ridx0, cidx0), (ridx1, cidx1))
    buf = ((rbuf0, cbuf0), (rbuf1, cbuf1))
    semi = (semi0, semi1)
    semg = (semg0, semg1)

    chunks = (
        [(er_hbm, ec_hbm, wid * (_E // _NW) + i * _C, 0)
         for i in range(_CH_MAIN)]
        + [(g1r_hbm, g1c_hbm, wid * (_EG // _NW) + i * _C, 1)
           for i in range(_CH_G)]
        + [(g2r_hbm, g2c_hbm, wid * (_EG // _NW) + i * _C, 2)
           for i in range(_CH_G)]
    )

    def stage_idx(k, slot):
        rref, cref, off, _ = chunks[k]
        c1 = pltpu.async_copy(rref.at[pl.ds(off, _C)], idx[slot][0],
                              semi[slot])
        c2 = pltpu.async_copy(cref.at[pl.ds(off, _C)], idx[slot][1],
                              semi[slot])
        return c1, c2

    def fire_gather(slot):
        c1 = pltpu.async_copy(a_hbm.at[idx[slot][0]], buf[slot][0],
                              semg[slot])
        c2 = pltpu.async_copy(a_hbm.at[idx[slot][1]], buf[slot][1],
                              semg[slot])
        return c1, c2

    pis = {0: stage_idx(0, 0), 1: stage_idx(1, 1)}
    pis[0][0].wait()
    pis[0][1].wait()
    pgs = {0: fire_gather(0)}

    zero = jnp.zeros((16,), jnp.float32)
    accs = [tuple(zero for _ in range(2 * _NACC)) for _ in range(3)]
    for k in range(_NCH):
        slot = k % 2
        if k + 1 < _NCH:
            pis[k + 1][0].wait()
            pis[k + 1][1].wait()
            pgs[k + 1] = fire_gather((k + 1) % 2)
        pgs[k][0].wait()
        pgs[k][1].wait()
        if k + 2 < _NCH:
            pis[k + 2] = stage_idx(k + 2, slot)
        rb, cb = buf[slot]

        def body(jj, carry, rb=rb, cb=cb):
            acc = list(carry)
            j0 = jj * _U
            for u in range(_U):
                r = rb[j0 + u]
                c = cb[j0 + u]
                i = 2 * (u % _NACC)
                acc[i] = acc[i] + r * c
                acc[i + 1] = acc[i + 1] + c
            return tuple(acc)

        s = chunks[k][3]
        accs[s] = lax.fori_loop(0, _C // _U, body, accs[s])

    for s in range(3):
        ad = accs[s][0] + accs[s][2]
        ac = accs[s][1] + accs[s][3]
        outv[pl.ds(32 * s, 16)] = ad
        outv[pl.ds(32 * s + 16, 16)] = ac
    pltpu.sync_copy(outv, out_hbm.at[pl.ds(wid * 96, 96)])


_edge_kernel = pl.kernel(
    _edge_body,
    out_type=jax.ShapeDtypeStruct((_NW * 96,), jnp.float32),
    mesh=plsc.VectorSubcoreMesh(core_axis_name="c", subcore_axis_name="s"),
    compiler_params=pltpu.CompilerParams(use_tc_tiling_on_sc=False),
    scratch_types=[
        pltpu.VMEM((_C,), jnp.int32),
        pltpu.VMEM((_C,), jnp.int32),
        pltpu.VMEM((_C,), jnp.int32),
        pltpu.VMEM((_C,), jnp.int32),
        pltpu.VMEM((_C, _K), jnp.float32),
        pltpu.VMEM((_C, _K), jnp.float32),
        pltpu.VMEM((_C, _K), jnp.float32),
        pltpu.VMEM((_C, _K), jnp.float32),
        pltpu.VMEM((96,), jnp.float32),
        pltpu.SemaphoreType.DMA,
        pltpu.SemaphoreType.DMA,
        pltpu.SemaphoreType.DMA,
        pltpu.SemaphoreType.DMA,
    ],
)


def kernel(features, edge_index, g1_edge_index, g2_edge_index, lamda, W, b):
    n = features.shape[0]
    k = W.shape[1]
    m = float(edge_index.shape[1])
    m1 = float(g1_edge_index.shape[1])
    m2 = float(g2_edge_index.shape[1])

    (a, csum, pooled_t, er, ec, g1r, g1c, g2r, g2c) = _tc_call(
        features, W, b.reshape(1, _K),
        edge_index, g1_edge_index, g2_edge_index)
    assignments = a
    cs = csum.reshape(_K)
    features_pooled = pooled_t.T

    partials = _edge_kernel(
        a, er, ec, g1r, g1c, g2r, g2c).reshape(_NW, 3, 2, _K)

    def set_stats(s):
        dot = jnp.sum(partials[:, s, 0, :])
        v = jnp.sum(partials[:, s, 1, :], axis=0)
        return dot, jnp.sum(v ** 2)

    s_main, vsq_main = set_stats(0)
    s_g1, vsq_g1 = set_stats(1)
    s_g2, vsq_g2 = set_stats(2)

    spectral_loss = -(s_main - vsq_main / (2.0 * m)) / (2.0 * m)
    q_g1 = (s_g1 - vsq_g1 / (2.0 * m1)) / (2.0 * m)
    q_g2 = (s_g2 - vsq_g2 / (2.0 * m2)) / (2.0 * m)
    edge_ratio = m / ((m1 + m2) / 2.0 + 1e-8)
    fairness_loss = edge_ratio * (-jnp.minimum(q_g1, q_g2))
    collapse_loss = (jnp.linalg.norm(cs) / n * jnp.sqrt(float(k)) - 1.0)

    total_loss = (spectral_loss + lamda * fairness_loss
                  + 0.1 * collapse_loss)
    return features_pooled, assignments, total_loss


# speedup vs baseline: 2.2973x; 1.0132x over previous
import functools

import jax
import jax.numpy as jnp
from jax import lax
from jax.experimental import pallas as pl
from jax.experimental.pallas import tpu as pltpu
from jax.experimental.pallas import tpu_sc as plsc

_N = 10000
_D = 128
_K = 16
_E = 320000
_EG = 160000

_SELU_ALPHA = 1.6732632423543772
_SELU_SCALE = 1.0507009873554805


_BLK = 1000
_NSTEP = _N // _BLK


def _tc_body(x_ref, w_ref, b_ref, e_ref, g1_ref, g2_ref,
             a_ref, csum_ref, pooled_ref,
             er_ref, ec_ref, g1r_ref, g1c_ref, g2r_ref, g2c_ref,
             atx_ref):
    i = pl.program_id(0)

    @pl.when(i == 0)
    def _():
        er_ref[...] = e_ref[0, :]
        ec_ref[...] = e_ref[1, :]
        g1r_ref[...] = g1_ref[0, :]
        g1c_ref[...] = g1_ref[1, :]
        g2r_ref[...] = g2_ref[0, :]
        g2c_ref[...] = g2_ref[1, :]
    x = x_ref[...]
    logits = jnp.dot(x, w_ref[...], preferred_element_type=jnp.float32)
    logits = logits + b_ref[...]
    mx = jnp.max(logits, axis=1, keepdims=True)
    e = jnp.exp(logits - mx)
    a = e / jnp.sum(e, axis=1, keepdims=True)
    a_ref[...] = a

    @pl.when(i == 0)
    def _():
        csum_ref[...] = jnp.zeros_like(csum_ref)
        atx_ref[...] = jnp.zeros_like(atx_ref)

    csum_ref[...] += jnp.sum(a, axis=0, keepdims=True)
    atx_ref[...] += lax.dot_general(
        x, a, (((0,), (0,)), ((), ())), preferred_element_type=jnp.float32)

    @pl.when(i == _NSTEP - 1)
    def _():
        pooled = atx_ref[...] / csum_ref[...]
        pooled_ref[...] = _SELU_SCALE * jnp.where(
            pooled > 0, pooled, _SELU_ALPHA * (jnp.exp(pooled) - 1.0))


_tc_call = pl.pallas_call(
    _tc_body,
    grid=(_NSTEP,),
    in_specs=[
        pl.BlockSpec((_BLK, _D), lambda i: (i, 0)),
        pl.BlockSpec((_D, _K), lambda i: (0, 0)),
        pl.BlockSpec((1, _K), lambda i: (0, 0)),
        pl.BlockSpec((2, _E), lambda i: (0, 0)),
        pl.BlockSpec((2, _EG), lambda i: (0, 0)),
        pl.BlockSpec((2, _EG), lambda i: (0, 0)),
    ],
    out_specs=[
        pl.BlockSpec((_BLK, _K), lambda i: (i, 0)),
        pl.BlockSpec((1, _K), lambda i: (0, 0)),
        pl.BlockSpec((_D, _K), lambda i: (0, 0)),
        pl.BlockSpec((_E,), lambda i: (0,)),
        pl.BlockSpec((_E,), lambda i: (0,)),
        pl.BlockSpec((_EG,), lambda i: (0,)),
        pl.BlockSpec((_EG,), lambda i: (0,)),
        pl.BlockSpec((_EG,), lambda i: (0,)),
        pl.BlockSpec((_EG,), lambda i: (0,)),
    ],
    out_shape=[
        jax.ShapeDtypeStruct((_N, _K), jnp.float32),
        jax.ShapeDtypeStruct((1, _K), jnp.float32),
        jax.ShapeDtypeStruct((_D, _K), jnp.float32),
        jax.ShapeDtypeStruct((_E,), jnp.int32),
        jax.ShapeDtypeStruct((_E,), jnp.int32),
        jax.ShapeDtypeStruct((_EG,), jnp.int32),
        jax.ShapeDtypeStruct((_EG,), jnp.int32),
        jax.ShapeDtypeStruct((_EG,), jnp.int32),
        jax.ShapeDtypeStruct((_EG,), jnp.int32),
    ],
    scratch_shapes=[pltpu.VMEM((_D, _K), jnp.float32)],
)


_NC = 2
_NS = 16
_NW = _NC * _NS
_C = 1000
_CH_MAIN = _E // _NW // _C
_CH_G = _EG // _NW // _C
_NCH = _CH_MAIN + 2 * _CH_G
_U = 20
_NACC = 2


def _edge_body(a_hbm, er_hbm, ec_hbm, g1r_hbm, g1c_hbm, g2r_hbm, g2c_hbm,
               out_hbm,
               ridx0, cidx0, ridx1, cidx1, ridx2, cidx2,
               rbuf0, cbuf0, rbuf1, cbuf1, rbuf2, cbuf2,
               outv, semi0, semi1, semi2, semg0, semg1, semg2):
    wid = lax.axis_index("s") * _NC + lax.axis_index("c")

    idx = ((ridx0, cidx0), (ridx1, cidx1), (ridx2, cidx2))
    buf = ((rbuf0, cbuf0), (rbuf1, cbuf1), (rbuf2, cbuf2))
    semi = (semi0, semi1, semi2)
    semg = (semg0, semg1, semg2)

    chunks = (
        [(er_hbm, ec_hbm, wid * (_E // _NW) + i * _C, 0)
         for i in range(_CH_MAIN)]
        + [(g1r_hbm, g1c_hbm, wid * (_EG // _NW) + i * _C, 1)
           for i in range(_CH_G)]
        + [(g2r_hbm, g2c_hbm, wid * (_EG // _NW) + i * _C, 2)
           for i in range(_CH_G)]
    )

    def stage_idx(k, slot):
        rref, cref, off, _ = chunks[k]
        c1 = pltpu.async_copy(rref.at[pl.ds(off, _C)], idx[slot][0],
                              semi[slot])
        c2 = pltpu.async_copy(cref.at[pl.ds(off, _C)], idx[slot][1],
                              semi[slot])
        return c1, c2

    def fire_gather(slot):
        c1 = pltpu.async_copy(a_hbm.at[idx[slot][0]], buf[slot][0],
                              semg[slot])
        c2 = pltpu.async_copy(a_hbm.at[idx[slot][1]], buf[slot][1],
                              semg[slot])
        return c1, c2

    pis = {k: stage_idx(k, k % 3) for k in range(3)}
    pgs = {}
    for k in range(2):
        pis[k][0].wait()
        pis[k][1].wait()
        pgs[k] = fire_gather(k % 3)

    zero = jnp.zeros((16,), jnp.float32)
    accs = [tuple(zero for _ in range(2 * _NACC)) for _ in range(3)]
    for k in range(_NCH):
        slot = k % 3
        if k + 2 < _NCH:
            pis[k + 2][0].wait()
            pis[k + 2][1].wait()
            pgs[k + 2] = fire_gather((k + 2) % 3)
        pgs[k][0].wait()
        pgs[k][1].wait()
        if k + 3 < _NCH:
            pis[k + 3] = stage_idx(k + 3, slot)
        rb, cb = buf[slot]

        def body(jj, carry, rb=rb, cb=cb):
            acc = list(carry)
            j0 = jj * _U
            for u in range(_U):
                r = rb[j0 + u]
                c = cb[j0 + u]
                i = 2 * (u % _NACC)
                acc[i] = acc[i] + r * c
                acc[i + 1] = acc[i + 1] + c
            return tuple(acc)

        s = chunks[k][3]
        accs[s] = lax.fori_loop(0, _C // _U, body, accs[s])

    for s in range(3):
        ad = accs[s][0] + accs[s][2]
        ac = accs[s][1] + accs[s][3]
        outv[pl.ds(32 * s, 16)] = ad
        outv[pl.ds(32 * s + 16, 16)] = ac
    pltpu.sync_copy(outv, out_hbm.at[pl.ds(wid * 96, 96)])


_edge_kernel = pl.kernel(
    _edge_body,
    out_type=jax.ShapeDtypeStruct((_NW * 96,), jnp.float32),
    mesh=plsc.VectorSubcoreMesh(core_axis_name="c", subcore_axis_name="s"),
    compiler_params=pltpu.CompilerParams(use_tc_tiling_on_sc=False),
    scratch_types=(
        [pltpu.VMEM((_C,), jnp.int32)] * 6
        + [pltpu.VMEM((_C, _K), jnp.float32)] * 6
        + [pltpu.VMEM((96,), jnp.float32)]
        + [pltpu.SemaphoreType.DMA] * 6
    ),
)


def kernel(features, edge_index, g1_edge_index, g2_edge_index, lamda, W, b):
    n = features.shape[0]
    k = W.shape[1]
    m = float(edge_index.shape[1])
    m1 = float(g1_edge_index.shape[1])
    m2 = float(g2_edge_index.shape[1])

    (a, csum, pooled_t, er, ec, g1r, g1c, g2r, g2c) = _tc_call(
        features, W, b.reshape(1, _K),
        edge_index, g1_edge_index, g2_edge_index)
    assignments = a
    cs = csum.reshape(_K)
    features_pooled = pooled_t.T

    partials = _edge_kernel(
        a, er, ec, g1r, g1c, g2r, g2c).reshape(_NW, 3, 2, _K)

    def set_stats(s):
        dot = jnp.sum(partials[:, s, 0, :])
        v = jnp.sum(partials[:, s, 1, :], axis=0)
        return dot, jnp.sum(v ** 2)

    s_main, vsq_main = set_stats(0)
    s_g1, vsq_g1 = set_stats(1)
    s_g2, vsq_g2 = set_stats(2)

    spectral_loss = -(s_main - vsq_main / (2.0 * m)) / (2.0 * m)
    q_g1 = (s_g1 - vsq_g1 / (2.0 * m1)) / (2.0 * m)
    q_g2 = (s_g2 - vsq_g2 / (2.0 * m2)) / (2.0 * m)
    edge_ratio = m / ((m1 + m2) / 2.0 + 1e-8)
    fairness_loss = edge_ratio * (-jnp.minimum(q_g1, q_g2))
    collapse_loss = (jnp.linalg.norm(cs) / n * jnp.sqrt(float(k)) - 1.0)

    total_loss = (spectral_loss + lamda * fairness_loss
                  + 0.1 * collapse_loss)
    return features_pooled, assignments, total_loss
